# Initial kernel scaffold; baseline (speedup 1.0000x reference)
import jax
import jax.numpy as jnp
from jax.experimental import pallas as pl


def kernel(x, position_embedding):
    raise NotImplementedError("write your pallas kernel here")


# speedup vs baseline: 14.4749x; 14.4749x over previous
import functools

import jax
import jax.numpy as jnp
from jax import lax
from jax.experimental import pallas as pl
from jax.experimental.pallas import tpu as pltpu
from jax.experimental.pallas import tpu_sc as plsc

_NUM_CORES = 2
_NUM_SUBCORES = 16
_NW = _NUM_CORES * _NUM_SUBCORES
_LANES = 16

_G = 2
_R = _G * 128
_NBUF = 2


def _body(xa_hbm, xb_hbm, pe_hbm, out_hbm, fa_v, xb_v, bi_v, fbuf, bbuf,
          gsem, wsem, *, rows_per_worker):
    wid = lax.axis_index("c") * _NUM_SUBCORES + lax.axis_index("s")
    nchunk = rows_per_worker // _R

    def do_chunk(cc, b):
        blk = wid * (rows_per_worker // 128) + cc * _G
        base = blk * 128

        pltpu.sync_copy(xa_hbm.at[pl.ds(blk, _G)], fa_v.at[b])
        pltpu.sync_copy(xb_hbm.at[pl.ds(blk, _G)], xb_v.at[b])

        for j in range(_G):
            for k in range(128 // _LANES):
                sl = pl.ds(k * _LANES, _LANES)
                a = fa_v[b, j, sl]
                bb = xb_v[b, j, sl]
                bi_v[b, j, sl] = jnp.where(a == 0, 0, bb - a + 1)

        handles = []
        for j in range(_G):
            dst = pl.ds(j * 128, 128)
            handles.append(pltpu.async_copy(
                pe_hbm.at[fa_v.at[b].at[j]], fbuf.at[b].at[dst], gsem))
            handles.append(pltpu.async_copy(
                pe_hbm.at[bi_v.at[b].at[j]], bbuf.at[b].at[dst], gsem))
        for h in handles:
            h.wait()

        pltpu.async_copy(fbuf.at[b], out_hbm.at[pl.ds(base, _R), 0],
                         wsem.at[b, 0])
        pltpu.async_copy(bbuf.at[b], out_hbm.at[pl.ds(base, _R), 1],
                         wsem.at[b, 1])

    def drain(b):
        pltpu.make_async_copy(
            fbuf.at[b], out_hbm.at[pl.ds(0, _R), 0], wsem.at[b, 0]).wait()
        pltpu.make_async_copy(
            bbuf.at[b], out_hbm.at[pl.ds(0, _R), 1], wsem.at[b, 1]).wait()

    def loop_body(c2, _):
        for b in range(_NBUF):
            @pl.when(c2 >= 1)
            def _():
                drain(b)

            do_chunk(c2 * _NBUF + b, b)
        return ()

    lax.fori_loop(0, nchunk // _NBUF, loop_body, ())

    for b in range(_NBUF):
        drain(b)


def kernel(x, position_embedding):
    s0, s1, _ = x.shape
    b_total = s0 * s1
    rows_per_worker = b_total // _NW
    xi = x.astype(jnp.int32)
    xa = xi[..., 0].reshape(-1, 128)
    xb = xi[..., 1].reshape(-1, 128)
    pe = position_embedding.astype(jnp.float32)

    mesh = plsc.VectorSubcoreMesh(
        core_axis_name="c", subcore_axis_name="s",
        num_cores=_NUM_CORES, num_subcores=_NUM_SUBCORES)
    k = pl.kernel(
        functools.partial(_body, rows_per_worker=rows_per_worker),
        out_type=jax.ShapeDtypeStruct((b_total, 2, 64), jnp.float32),
        mesh=mesh,
        compiler_params=pltpu.CompilerParams(use_tc_tiling_on_sc=False),
        scratch_types=[
            pltpu.VMEM((_NBUF, _G, 128), jnp.int32),
            pltpu.VMEM((_NBUF, _G, 128), jnp.int32),
            pltpu.VMEM((_NBUF, _G, 128), jnp.int32),
            pltpu.VMEM((_NBUF, _R, 64), jnp.float32),
            pltpu.VMEM((_NBUF, _R, 64), jnp.float32),
            pltpu.SemaphoreType.DMA,
            pltpu.SemaphoreType.DMA((_NBUF, 2)),
        ],
    )
    out = k(xa, xb, pe)
    return out.reshape(s0, s1, 128)


# speedup vs baseline: 17.4186x; 1.2034x over previous
import functools

import jax
import jax.numpy as jnp
from jax import lax
from jax.experimental import pallas as pl
from jax.experimental.pallas import tpu as pltpu
from jax.experimental.pallas import tpu_sc as plsc

_NUM_CORES = 2
_NUM_SUBCORES = 16
_NW = _NUM_CORES * _NUM_SUBCORES
_LANES = 16

_G = 2
_R = _G * 128
_NBUF = 2


def _body(xab_hbm, pe_hbm, out_hbm, x_v, bi_v, fbuf, bbuf,
          xsem, gsem, wsem, *, rows_per_worker):
    wid = lax.axis_index("c") * _NUM_SUBCORES + lax.axis_index("s")
    nchunk = rows_per_worker // _R
    blk0 = wid * (rows_per_worker // 128)

    def fire_xload(cc, b):
        pltpu.async_copy(xab_hbm.at[pl.ds(blk0 + cc * _G, _G)], x_v.at[b],
                         xsem.at[b])

    def wait_xload(b):
        pltpu.make_async_copy(xab_hbm.at[pl.ds(0, _G)], x_v.at[b],
                              xsem.at[b]).wait()

    def compute_bwd(b):
        for j in range(_G):
            for k in range(128 // _LANES):
                sl = pl.ds(k * _LANES, _LANES)
                a = x_v[b, j, 0, sl]
                bb = x_v[b, j, 1, sl]
                bi_v[b, j, sl] = jnp.where(a == 0, 0, bb - a + 1)

    def fire_gathers(b):
        for j in range(_G):
            dst = pl.ds(j * 128, 128)
            pltpu.async_copy(pe_hbm.at[x_v.at[b].at[j].at[0]],
                             fbuf.at[b].at[dst], gsem.at[b])
            pltpu.async_copy(pe_hbm.at[bi_v.at[b].at[j]],
                             bbuf.at[b].at[dst], gsem.at[b])

    def drain_gathers(b):
        pltpu.make_async_copy(pe_hbm.at[pl.ds(0, _R)], fbuf.at[b],
                              gsem.at[b]).wait()
        pltpu.make_async_copy(pe_hbm.at[pl.ds(0, _R)], bbuf.at[b],
                              gsem.at[b]).wait()

    def fire_writeback(cc, b):
        base = (blk0 + cc * _G) * 128
        pltpu.async_copy(fbuf.at[b], out_hbm.at[pl.ds(base, _R), 0],
                         wsem.at[b, 0])
        pltpu.async_copy(bbuf.at[b], out_hbm.at[pl.ds(base, _R), 1],
                         wsem.at[b, 1])

    def drain_writeback(b):
        pltpu.make_async_copy(fbuf.at[b], out_hbm.at[pl.ds(0, _R), 0],
                              wsem.at[b, 0]).wait()
        pltpu.make_async_copy(bbuf.at[b], out_hbm.at[pl.ds(0, _R), 1],
                              wsem.at[b, 1]).wait()

    fire_xload(0, 0)

    def loop_body(c2, _):
        for b in range(_NBUF):
            cc = c2 * _NBUF + b
            wait_xload(b)
            compute_bwd(b)
            if b == 0:
                @pl.when(c2 == 0)
                def _():
                    fire_xload(1, 1)

                @pl.when(c2 >= 1)
                def _():
                    drain_gathers(1)
                    fire_xload(cc + 1, 1)
                    fire_writeback(cc - 1, 1)
                    drain_writeback(0)
            else:
                drain_gathers(0)

                @pl.when(c2 < nchunk // _NBUF - 1)
                def _():
                    fire_xload(cc + 1, 0)

                fire_writeback(cc - 1, 0)

                @pl.when(c2 >= 1)
                def _():
                    drain_writeback(1)

            fire_gathers(b)
        return ()

    lax.fori_loop(0, nchunk // _NBUF, loop_body, ())

    last = nchunk - 1
    drain_gathers(last % _NBUF)
    fire_writeback(last, last % _NBUF)
    for b in range(_NBUF):
        drain_writeback(b)


def kernel(x, position_embedding):
    s0, s1, _ = x.shape
    b_total = s0 * s1
    rows_per_worker = b_total // _NW
    xi = x.astype(jnp.int32)
    xab = xi.reshape(-1, 128, 2).transpose(0, 2, 1)
    pe = position_embedding.astype(jnp.float32)

    mesh = plsc.VectorSubcoreMesh(
        core_axis_name="c", subcore_axis_name="s",
        num_cores=_NUM_CORES, num_subcores=_NUM_SUBCORES)
    k = pl.kernel(
        functools.partial(_body, rows_per_worker=rows_per_worker),
        out_type=jax.ShapeDtypeStruct((b_total, 2, 64), jnp.float32),
        mesh=mesh,
        compiler_params=pltpu.CompilerParams(use_tc_tiling_on_sc=False),
        scratch_types=[
            pltpu.VMEM((_NBUF, _G, 2, 128), jnp.int32),
            pltpu.VMEM((_NBUF, _G, 128), jnp.int32),
            pltpu.VMEM((_NBUF, _R, 64), jnp.float32),
            pltpu.VMEM((_NBUF, _R, 64), jnp.float32),
            pltpu.SemaphoreType.DMA((_NBUF,)),
            pltpu.SemaphoreType.DMA((_NBUF,)),
            pltpu.SemaphoreType.DMA((_NBUF, 2)),
        ],
    )
    out = k(xab, pe)
    return out.reshape(s0, s1, 128)
